# baseline (speedup 1.0000x reference)
import functools

import jax
import jax.numpy as jnp
from jax import lax
from jax.experimental import pallas as pl
from jax.experimental.pallas import tpu as pltpu
from jax.experimental.pallas import tpu_sc as plsc

N = 10000
D = 128
H = 8
DH = 16
ED = 16
FFN = 512
E_FULL = 320000
EG = 160000

F32 = jnp.float32

CH = 128
NCH_HALF = EG // CH
NK_GATHER = (NCH_HALF + 15) // 16
CS = 64
NCS_HALF = EG // CS
NCS_ALL = E_FULL // CS
NK_SCAT = (NCS_ALL + 15) // 16
ROW_SPAN = 624
ROW_SPAN_LAST = N - 15 * ROW_SPAN

BE = 2000
NBLK_G = EG // BE


def _dot(a, b, dims):
    return lax.dot_general(a, b, (dims, ((), ())),
                           preferred_element_type=F32)


def _sel_128x8():
    r = lax.broadcasted_iota(jnp.int32, (D, H), 0) // DH
    c = lax.broadcasted_iota(jnp.int32, (D, H), 1)
    return (r == c).astype(F32)


def _node_pro_body(x, wq, bq, wk, bk, wv, bv, gw, gb, ga, qh, kh, vh):
    xv = x[...]
    mean = jnp.mean(xv, axis=0, keepdims=True)
    sub = xv - ga[...] * mean
    var = jnp.mean(sub * sub, axis=0, keepdims=True)
    y = gw[...] * sub * lax.rsqrt(var + 1e-5) + gb[...]
    qh[...] = _dot(y, wq[...], ((1,), (1,))) + bq[...]
    kh[...] = _dot(y, wk[...], ((1,), (1,))) + bk[...]
    vh[...] = _dot(y, wv[...], ((1,), (1,))) + bv[...]


def _edge_pro_body(e, lw, lb, wpe, bpe, enorm, proj):
    ev = e[...]
    m = jnp.mean(ev, axis=1, keepdims=True)
    v = jnp.mean((ev - m) ** 2, axis=1, keepdims=True)
    en = lw[...] * (ev - m) * lax.rsqrt(v + 1e-5) + lb[...]
    enorm[...] = en
    proj[...] = _dot(en, wpe[...], ((1,), (1,))) + bpe[...]


def _score_g_body(ks, qd, adj, rel, proj, wap, bap, eout, exw_o):
    sel = _sel_128x8()
    t = ks[...] * qd[...]
    dot = _dot(t, sel, ((1,), (0,)))
    score = dot * (adj[...] * 0.25) + rel[...]
    eout[...] = _dot(score, wap[...], ((1,), (1,))) + bap[...]
    score = score + proj[...]
    ex = jnp.exp(jnp.clip(score, -5.0, 5.0))
    exw_o[...] = _dot(ex, sel, ((1,), (1,)))


def _score_n_body(ks, qd, adj, rel, exw_o):
    sel = _sel_128x8()
    t = ks[...] * qd[...]
    dot = _dot(t, sel, ((1,), (0,)))
    score = dot * (adj[...] * 0.25) + rel[...]
    ex = jnp.exp(jnp.clip(score, -5.0, 5.0))
    exw_o[...] = _dot(ex, sel, ((1,), (1,)))


def _node_epi1_body(x, wv, den, wo, bo, ga, x2, mu, var):
    denw = den[...] + 1e-16
    wvn_ = wv[...] / denw
    h_out = _dot(wvn_, wo[...], ((1,), (1,))) + bo[...]
    x2v = x[...] + h_out
    x2[...] = x2v
    m = jnp.mean(x2v, axis=0, keepdims=True)
    sub = x2v - ga[...] * m
    mu[...] = m
    var[...] = jnp.mean(sub * sub, axis=0, keepdims=True)


def _node_epi2_body(x2, mu, var, gw, gb, ga, w1, b1, w2, b2, x3):
    x2v = x2[...]
    sub = x2v - ga[...] * mu[...]
    y2 = gw[...] * sub * lax.rsqrt(var[...] + 1e-5) + gb[...]
    h = jax.nn.gelu(_dot(y2, w1[...], ((1,), (1,))) + b1[...])
    x3[...] = x2v + _dot(h, w2[...], ((1,), (1,))) + b2[...]


def _edge_epi_body(e, enorm, eout, woe, boe, lw, lb, w1, b1, w2, b2, e3):
    e_att = _dot(eout[...] + enorm[...], woe[...], ((1,), (1,))) + boe[...]
    e2 = e[...] + e_att
    m = jnp.mean(e2, axis=1, keepdims=True)
    v = jnp.mean((e2 - m) ** 2, axis=1, keepdims=True)
    en2 = lw[...] * (e2 - m) * lax.rsqrt(v + 1e-5) + lb[...]
    h = jax.nn.gelu(_dot(en2, w1[...], ((1,), (1,))) + b1[...])
    e3[...] = e2 + _dot(h, w2[...], ((1,), (1,))) + b2[...]


_MESH = plsc.VectorSubcoreMesh(core_axis_name="c", subcore_axis_name="s")


NCH_CORE = NCH_HALF // 2
NK_GATHER2 = (NCH_CORE + 15) // 16


def _make_gather(half_base):
    @functools.partial(
        pl.kernel,
        out_type=[jax.ShapeDtypeStruct((EG, D), F32) for _ in range(2)],
        mesh=_MESH,
        scratch_types=[
            [pltpu.VMEM((CH,), jnp.int32) for _ in range(2)],
            [pltpu.VMEM((CH,), jnp.int32) for _ in range(2)],
            [pltpu.VMEM((CH, D), F32) for _ in range(2)],
            [pltpu.VMEM((CH, D), F32) for _ in range(2)],
            [pltpu.SemaphoreType.DMA for _ in range(2)],
            [pltpu.SemaphoreType.DMA for _ in range(2)],
            [pltpu.SemaphoreType.DMA for _ in range(2)],
        ],
    )
    def gather(kh, qh, src, dst, ks_o, qd_o,
               idx_s, idx_d, rk, rq, sem_i, sem_g, sem_w):
        c = lax.axis_index("c")
        s = lax.axis_index("s")

        def start_idx(k, b):
            tl = s + 16 * k

            @pl.when(tl < NCH_CORE)
            def _():
                base = half_base + (c * NCH_CORE + tl) * CH
                pltpu.async_copy(src.at[pl.ds(base, CH)], idx_s[b], sem_i[b])
                pltpu.async_copy(dst.at[pl.ds(base, CH)], idx_d[b], sem_i[b])

        def drain_w(b):
            pltpu.make_async_copy(rk[b], ks_o.at[pl.ds(0, CH)], sem_w[b]).wait()
            pltpu.make_async_copy(rq[b], qd_o.at[pl.ds(0, CH)], sem_w[b]).wait()

        start_idx(0, 0)

        def step(k, b):
            tl = s + 16 * k

            @pl.when(tl < NCH_CORE)
            def _():
                loc = (c * NCH_CORE + tl) * CH
                pltpu.make_async_copy(src.at[pl.ds(0, CH)], idx_s[b], sem_i[b]).wait()
                pltpu.make_async_copy(dst.at[pl.ds(0, CH)], idx_d[b], sem_i[b]).wait()

                @pl.when(k >= 2)
                def _():
                    drain_w(b)

                pltpu.async_copy(kh.at[idx_s[b]], rk[b], sem_g[b])
                pltpu.async_copy(qh.at[idx_d[b]], rq[b], sem_g[b])
                start_idx(k + 1, 1 - b)
                pltpu.make_async_copy(kh.at[idx_s[b]], rk[b], sem_g[b]).wait()
                pltpu.make_async_copy(qh.at[idx_d[b]], rq[b], sem_g[b]).wait()
                pltpu.async_copy(rk[b], ks_o.at[pl.ds(loc, CH)], sem_w[b])
                pltpu.async_copy(rq[b], qd_o.at[pl.ds(loc, CH)], sem_w[b])

        def body(k2, z):
            step(2 * k2, 0)
            step(2 * k2 + 1, 1)
            return z

        lax.fori_loop(0, (NK_GATHER2 + 1) // 2, body, 0)

        def drain_tail(k):
            b = k % 2
            tl = s + 16 * k
            tl2 = s + 16 * (k + 2)

            @pl.when((tl < NCH_CORE) & (tl2 >= NCH_CORE))
            def _():
                drain_w(b)

        drain_tail(NK_GATHER2 - 3)
        drain_tail(NK_GATHER2 - 2)
        drain_tail(NK_GATHER2 - 1)

    return gather


_sc_gather_g = _make_gather(0)
_sc_gather_n = _make_gather(EG)


@functools.partial(
    pl.kernel,
    out_type=[
        jax.ShapeDtypeStruct((N, D), F32),
        jax.ShapeDtypeStruct((N, D), F32),
    ],
    mesh=_MESH,
    scratch_types=[
        [pltpu.VMEM((CS,), jnp.int32) for _ in range(2)],
        [pltpu.VMEM((CS,), jnp.int32) for _ in range(2)],
        [pltpu.VMEM((CS, D), F32) for _ in range(2)],
        [pltpu.VMEM((CS, D), F32) for _ in range(2)],
        pltpu.VMEM_SHARED((N, D), F32),
        [pltpu.SemaphoreType.DMA for _ in range(2)],
        [pltpu.SemaphoreType.DMA for _ in range(2)],
        [pltpu.SemaphoreType.DMA for _ in range(2)],
        [pltpu.SemaphoreType.DMA for _ in range(2)],
    ],
)
def _sc_scatter(dst, src, exwg, exwn, vh, z128,
                wv_o, den_o,
                idx_d, idx_s, exb, vb, acc, sem_i, sem_r, sem_g, sem_s):
    c = lax.axis_index("c")
    s = lax.axis_index("s")

    @pl.when(s < 15)
    def _():
        pltpu.sync_copy(z128.at[pl.ds(s * ROW_SPAN, ROW_SPAN)],
                        acc.at[pl.ds(s * ROW_SPAN, ROW_SPAN)])

    @pl.when(s == 15)
    def _():
        pltpu.sync_copy(z128.at[pl.ds(15 * ROW_SPAN, ROW_SPAN_LAST)],
                        acc.at[pl.ds(15 * ROW_SPAN, ROW_SPAN_LAST)])

    plsc.subcore_barrier()

    def run(with_v):
        sbuf = vb if with_v else exb

        def start_loads(k, b):
            t = s + 16 * k

            @pl.when(t < NCS_ALL)
            def _():
                pltpu.async_copy(dst.at[pl.ds(t * CS, CS)], idx_d[b], sem_i[b])
                if with_v:
                    pltpu.async_copy(src.at[pl.ds(t * CS, CS)], idx_s[b], sem_i[b])

                @pl.when(t < NCS_HALF)
                def _():
                    pltpu.async_copy(exwg.at[pl.ds(t * CS, CS)], exb[b], sem_r[b])

                @pl.when(t >= NCS_HALF)
                def _():
                    pltpu.async_copy(exwn.at[pl.ds((t - NCS_HALF) * CS, CS)],
                                     exb[b], sem_r[b])

        start_loads(0, 0)

        def step(k, b):
            t = s + 16 * k

            @pl.when(t < NCS_ALL)
            def _():
                pltpu.make_async_copy(dst.at[pl.ds(0, CS)], idx_d[b], sem_i[b]).wait()
                if with_v:
                    pltpu.make_async_copy(src.at[pl.ds(0, CS)], idx_s[b],
                                          sem_i[b]).wait()
                pltpu.make_async_copy(exwg.at[pl.ds(0, CS)], exb[b], sem_r[b]).wait()
                if with_v:
                    pltpu.async_copy(vh.at[idx_s[b]], vb[b], sem_g[b])

                @pl.when(k >= 1)
                def _():
                    pltpu.make_async_copy(sbuf[1 - b], acc.at[idx_d[1 - b]],
                                          sem_s[1 - b]).wait()

                start_loads(k + 1, 1 - b)

                if with_v:
                    pltpu.make_async_copy(vh.at[idx_s[b]], vb[b], sem_g[b]).wait()

                    def mul_body(e, z):
                        for h in range(H):
                            sl = pl.ds(16 * h, 16)
                            vb[b][e, sl] = vb[b][e, sl] * exb[b][e, sl]
                        return z

                    lax.fori_loop(0, CS, mul_body, 0)

                pltpu.async_copy(sbuf[b], acc.at[idx_d[b]], sem_s[b], add=True)

        def body(k2, z):
            step(2 * k2, 0)
            step(2 * k2 + 1, 1)
            return z

        lax.fori_loop(0, (NK_SCAT + 1) // 2, body, 0)

        t_last = s + 16 * (NK_SCAT - 1)
        b_last = (NK_SCAT - 1) % 2

        @pl.when(t_last < NCS_ALL)
        def _():
            pltpu.make_async_copy(sbuf[b_last], acc.at[idx_d[b_last]],
                                  sem_s[b_last]).wait()

        @pl.when(t_last >= NCS_ALL)
        def _():
            pltpu.make_async_copy(sbuf[1 - b_last], acc.at[idx_d[1 - b_last]],
                                  sem_s[1 - b_last]).wait()

    @pl.when(c == 0)
    def _():
        run(True)

    @pl.when(c == 1)
    def _():
        run(False)

    plsc.subcore_barrier()

    def flush(out):
        @pl.when(s < 15)
        def _():
            pltpu.sync_copy(acc.at[pl.ds(s * ROW_SPAN, ROW_SPAN)],
                            out.at[pl.ds(s * ROW_SPAN, ROW_SPAN)])

        @pl.when(s == 15)
        def _():
            pltpu.sync_copy(acc.at[pl.ds(15 * ROW_SPAN, ROW_SPAN_LAST)],
                            out.at[pl.ds(15 * ROW_SPAN, ROW_SPAN_LAST)])

    @pl.when(c == 0)
    def _():
        flush(wv_o)

    @pl.when(c == 1)
    def _():
        flush(den_o)


def _full(shape):
    nd = len(shape)
    return pl.BlockSpec(shape, lambda i: (0,) * nd)


def kernel(x, e, adj2, rel_pos_3d, edge_index_full, g_eids, params):
    p = params
    src = edge_index_full[0].astype(jnp.int32)
    dst = edge_index_full[1].astype(jnp.int32)

    r1 = lambda a: a.reshape(1, -1)

    qh, kh, vh = pl.pallas_call(
        _node_pro_body,
        out_shape=[jax.ShapeDtypeStruct((N, D), F32) for _ in range(3)],
    )(x, p['Wq'], r1(p['bq']), p['Wk'], r1(p['bk']), p['Wv'], r1(p['bv']),
      r1(p['gn1_w']), r1(p['gn1_b']), r1(p['gn1_a']))

    enorm, proj_e = pl.pallas_call(
        _edge_pro_body,
        grid=(NBLK_G,),
        in_specs=[pl.BlockSpec((BE, ED), lambda i: (i, 0)),
                  _full((1, ED)), _full((1, ED)),
                  _full((H, ED)), _full((1, H))],
        out_specs=[pl.BlockSpec((BE, ED), lambda i: (i, 0)),
                   pl.BlockSpec((BE, H), lambda i: (i, 0))],
        out_shape=[jax.ShapeDtypeStruct((EG, ED), F32),
                   jax.ShapeDtypeStruct((EG, H), F32)],
    )(e, r1(p['ln1e_w']), r1(p['ln1e_b']), p['Wpe'], r1(p['bpe']))

    ksg, qdg = _sc_gather_g(kh, qh, src, dst)
    ksn, qdn = _sc_gather_n(kh, qh, src, dst)

    eout, exwg = pl.pallas_call(
        _score_g_body,
        grid=(NBLK_G,),
        in_specs=[pl.BlockSpec((BE, D), lambda i: (i, 0)),
                  pl.BlockSpec((BE, D), lambda i: (i, 0)),
                  pl.BlockSpec((BE, 1), lambda i: (i, 0)),
                  pl.BlockSpec((BE, H), lambda i: (i, 0)),
                  pl.BlockSpec((BE, H), lambda i: (i, 0)),
                  _full((ED, H)), _full((1, ED))],
        out_specs=[pl.BlockSpec((BE, ED), lambda i: (i, 0)),
                   pl.BlockSpec((BE, D), lambda i: (i, 0))],
        out_shape=[jax.ShapeDtypeStruct((EG, ED), F32),
                   jax.ShapeDtypeStruct((EG, D), F32)],
    )(ksg, qdg, adj2[:EG], rel_pos_3d[:EG], proj_e,
      p['Wap'], r1(p['bap']))

    exwn = pl.pallas_call(
        _score_n_body,
        grid=(NBLK_G,),
        in_specs=[pl.BlockSpec((BE, D), lambda i: (i, 0)),
                  pl.BlockSpec((BE, D), lambda i: (i, 0)),
                  pl.BlockSpec((BE, 1), lambda i: (i, 0)),
                  pl.BlockSpec((BE, H), lambda i: (i, 0))],
        out_specs=pl.BlockSpec((BE, D), lambda i: (i, 0)),
        out_shape=jax.ShapeDtypeStruct((EG, D), F32),
    )(ksn, qdn, adj2[EG:], rel_pos_3d[EG:])

    z128 = jnp.zeros((N, D), F32)
    wv, den = _sc_scatter(dst, src, exwg, exwn, vh, z128)

    x2, mu, var = pl.pallas_call(
        _node_epi1_body,
        out_shape=[jax.ShapeDtypeStruct((N, D), F32),
                   jax.ShapeDtypeStruct((1, D), F32),
                   jax.ShapeDtypeStruct((1, D), F32)],
    )(x, wv, den, p['Wo'], r1(p['bo']), r1(p['gn2_a']))

    x3 = pl.pallas_call(
        _node_epi2_body,
        grid=(5,),
        in_specs=[pl.BlockSpec((BE, D), lambda i: (i, 0)),
                  _full((1, D)), _full((1, D)),
                  _full((1, D)), _full((1, D)), _full((1, D)),
                  _full((FFN, D)), _full((1, FFN)),
                  _full((D, FFN)), _full((1, D))],
        out_specs=pl.BlockSpec((BE, D), lambda i: (i, 0)),
        out_shape=jax.ShapeDtypeStruct((N, D), F32),
    )(x2, mu, var, r1(p['gn2_w']), r1(p['gn2_b']), r1(p['gn2_a']),
      p['Wh1'], r1(p['bh1']), p['Wh2'], r1(p['bh2']))

    e3 = pl.pallas_call(
        _edge_epi_body,
        grid=(NBLK_G,),
        in_specs=[pl.BlockSpec((BE, ED), lambda i: (i, 0)),
                  pl.BlockSpec((BE, ED), lambda i: (i, 0)),
                  pl.BlockSpec((BE, ED), lambda i: (i, 0)),
                  _full((ED, ED)), _full((1, ED)),
                  _full((1, ED)), _full((1, ED)),
                  _full((FFN, ED)), _full((1, FFN)),
                  _full((ED, FFN)), _full((1, ED))],
        out_specs=pl.BlockSpec((BE, ED), lambda i: (i, 0)),
        out_shape=jax.ShapeDtypeStruct((EG, ED), F32),
    )(e, enorm, eout, p['Woe'], r1(p['boe']),
      r1(p['ln2e_w']), r1(p['ln2e_b']),
      p['We1'], r1(p['be1']), p['We2'], r1(p['be2']))

    return (x3, e3)


# speedup vs baseline: 1.0794x; 1.0794x over previous
import functools

import jax
import jax.numpy as jnp
from jax import lax
from jax.experimental import pallas as pl
from jax.experimental.pallas import tpu as pltpu
from jax.experimental.pallas import tpu_sc as plsc

N = 10000
D = 128
H = 8
DH = 16
ED = 16
FFN = 512
E_FULL = 320000
EG = 160000

F32 = jnp.float32

CH = 128
NCH_HALF = EG // CH
NK_GATHER = (NCH_HALF + 15) // 16
NCH_ALL = E_FULL // CH
NK_SCAT = (NCH_ALL + 15) // 16
ROW_SPAN = 624
ROW_SPAN_LAST = N - 15 * ROW_SPAN

BE = 2000
NBLK_G = EG // BE


def _dot(a, b, dims):
    return lax.dot_general(a, b, (dims, ((), ())),
                           preferred_element_type=F32)


def _sel_128x8():
    r = lax.broadcasted_iota(jnp.int32, (D, H), 0) // DH
    c = lax.broadcasted_iota(jnp.int32, (D, H), 1)
    return (r == c).astype(F32)


def _node_pro_body(x, wq, bq, wk, bk, wv, bv, gw, gb, ga, qh, kh, vh):
    xv = x[...]
    mean = jnp.mean(xv, axis=0, keepdims=True)
    sub = xv - ga[...] * mean
    var = jnp.mean(sub * sub, axis=0, keepdims=True)
    y = gw[...] * sub * lax.rsqrt(var + 1e-5) + gb[...]
    qh[...] = _dot(y, wq[...], ((1,), (1,))) + bq[...]
    kh[...] = _dot(y, wk[...], ((1,), (1,))) + bk[...]
    vh[...] = _dot(y, wv[...], ((1,), (1,))) + bv[...]


def _edge_pro_body(e, lw, lb, wpe, bpe, enorm, proj):
    ev = e[...]
    m = jnp.mean(ev, axis=1, keepdims=True)
    v = jnp.mean((ev - m) ** 2, axis=1, keepdims=True)
    en = lw[...] * (ev - m) * lax.rsqrt(v + 1e-5) + lb[...]
    enorm[...] = en
    proj[...] = _dot(en, wpe[...], ((1,), (1,))) + bpe[...]


def _score_g_body(ks, qd, vs, adj, rel, proj, wap, bap, eout, exw_o, p):
    sel = _sel_128x8()
    t = ks[...] * qd[...]
    dot = _dot(t, sel, ((1,), (0,)))
    score = dot * (adj[...] * 0.25) + rel[...]
    eout[...] = _dot(score, wap[...], ((1,), (1,))) + bap[...]
    score = score + proj[...]
    ex = jnp.exp(jnp.clip(score, -5.0, 5.0))
    exw = _dot(ex, sel, ((1,), (1,)))
    exw_o[...] = exw
    p[...] = vs[...] * exw


def _score_n_body(ks, qd, vs, adj, rel, exw_o, p):
    sel = _sel_128x8()
    t = ks[...] * qd[...]
    dot = _dot(t, sel, ((1,), (0,)))
    score = dot * (adj[...] * 0.25) + rel[...]
    ex = jnp.exp(jnp.clip(score, -5.0, 5.0))
    exw = _dot(ex, sel, ((1,), (1,)))
    exw_o[...] = exw
    p[...] = vs[...] * exw


def _node_epi1_body(x, wv, den, wo, bo, ga, x2, mu, var):
    denw = den[...] + 1e-16
    wvn_ = wv[...] / denw
    h_out = _dot(wvn_, wo[...], ((1,), (1,))) + bo[...]
    x2v = x[...] + h_out
    x2[...] = x2v
    m = jnp.mean(x2v, axis=0, keepdims=True)
    sub = x2v - ga[...] * m
    mu[...] = m
    var[...] = jnp.mean(sub * sub, axis=0, keepdims=True)


def _node_epi2_body(x2, mu, var, gw, gb, ga, w1, b1, w2, b2, x3):
    x2v = x2[...]
    sub = x2v - ga[...] * mu[...]
    y2 = gw[...] * sub * lax.rsqrt(var[...] + 1e-5) + gb[...]
    h = jax.nn.gelu(_dot(y2, w1[...], ((1,), (1,))) + b1[...])
    x3[...] = x2v + _dot(h, w2[...], ((1,), (1,))) + b2[...]


def _edge_epi_body(e, enorm, eout, woe, boe, lw, lb, w1, b1, w2, b2, e3):
    e_att = _dot(eout[...] + enorm[...], woe[...], ((1,), (1,))) + boe[...]
    e2 = e[...] + e_att
    m = jnp.mean(e2, axis=1, keepdims=True)
    v = jnp.mean((e2 - m) ** 2, axis=1, keepdims=True)
    en2 = lw[...] * (e2 - m) * lax.rsqrt(v + 1e-5) + lb[...]
    h = jax.nn.gelu(_dot(en2, w1[...], ((1,), (1,))) + b1[...])
    e3[...] = e2 + _dot(h, w2[...], ((1,), (1,))) + b2[...]


_MESH = plsc.VectorSubcoreMesh(core_axis_name="c", subcore_axis_name="s")


NCH_CORE = NCH_HALF // 2
NK_GATHER2 = (NCH_CORE + 15) // 16


def _make_gather(half_base):
    @functools.partial(
        pl.kernel,
        out_type=[jax.ShapeDtypeStruct((EG, D), F32) for _ in range(3)],
        mesh=_MESH,
        scratch_types=[
            [pltpu.VMEM((CH,), jnp.int32) for _ in range(2)],
            [pltpu.VMEM((CH,), jnp.int32) for _ in range(2)],
            [pltpu.VMEM((CH, D), F32) for _ in range(2)],
            [pltpu.VMEM((CH, D), F32) for _ in range(2)],
            [pltpu.VMEM((CH, D), F32) for _ in range(2)],
            [pltpu.SemaphoreType.DMA for _ in range(2)],
            [pltpu.SemaphoreType.DMA for _ in range(2)],
            [pltpu.SemaphoreType.DMA for _ in range(2)],
        ],
    )
    def gather(kh, qh, vh, src, dst, ks_o, qd_o, vs_o,
               idx_s, idx_d, rk, rq, rv, sem_i, sem_g, sem_w):
        c = lax.axis_index("c")
        s = lax.axis_index("s")

        def start_idx(k, b):
            tl = s + 16 * k

            @pl.when(tl < NCH_CORE)
            def _():
                base = half_base + (c * NCH_CORE + tl) * CH
                pltpu.async_copy(src.at[pl.ds(base, CH)], idx_s[b], sem_i[b])
                pltpu.async_copy(dst.at[pl.ds(base, CH)], idx_d[b], sem_i[b])

        def drain_w(b):
            pltpu.make_async_copy(rk[b], ks_o.at[pl.ds(0, CH)], sem_w[b]).wait()
            pltpu.make_async_copy(rq[b], qd_o.at[pl.ds(0, CH)], sem_w[b]).wait()
            pltpu.make_async_copy(rv[b], vs_o.at[pl.ds(0, CH)], sem_w[b]).wait()

        start_idx(0, 0)

        def step(k, b):
            tl = s + 16 * k

            @pl.when(tl < NCH_CORE)
            def _():
                loc = (c * NCH_CORE + tl) * CH
                pltpu.make_async_copy(src.at[pl.ds(0, CH)], idx_s[b], sem_i[b]).wait()
                pltpu.make_async_copy(dst.at[pl.ds(0, CH)], idx_d[b], sem_i[b]).wait()

                @pl.when(k >= 2)
                def _():
                    drain_w(b)

                pltpu.async_copy(kh.at[idx_s[b]], rk[b], sem_g[b])
                pltpu.async_copy(qh.at[idx_d[b]], rq[b], sem_g[b])
                pltpu.async_copy(vh.at[idx_s[b]], rv[b], sem_g[b])
                start_idx(k + 1, 1 - b)
                pltpu.make_async_copy(kh.at[idx_s[b]], rk[b], sem_g[b]).wait()
                pltpu.make_async_copy(qh.at[idx_d[b]], rq[b], sem_g[b]).wait()
                pltpu.make_async_copy(vh.at[idx_s[b]], rv[b], sem_g[b]).wait()
                pltpu.async_copy(rk[b], ks_o.at[pl.ds(loc, CH)], sem_w[b])
                pltpu.async_copy(rq[b], qd_o.at[pl.ds(loc, CH)], sem_w[b])
                pltpu.async_copy(rv[b], vs_o.at[pl.ds(loc, CH)], sem_w[b])

        def body(k2, z):
            step(2 * k2, 0)
            step(2 * k2 + 1, 1)
            return z

        lax.fori_loop(0, (NK_GATHER2 + 1) // 2, body, 0)

        def drain_tail(k):
            b = k % 2
            tl = s + 16 * k
            tl2 = s + 16 * (k + 2)

            @pl.when((tl < NCH_CORE) & (tl2 >= NCH_CORE))
            def _():
                drain_w(b)

        drain_tail(NK_GATHER2 - 3)
        drain_tail(NK_GATHER2 - 2)
        drain_tail(NK_GATHER2 - 1)

    return gather


_sc_gather_g = _make_gather(0)
_sc_gather_n = _make_gather(EG)


@functools.partial(
    pl.kernel,
    out_type=[
        jax.ShapeDtypeStruct((N, D), F32),
        jax.ShapeDtypeStruct((N, D), F32),
    ],
    mesh=_MESH,
    scratch_types=[
        [pltpu.VMEM((CH,), jnp.int32) for _ in range(2)],
        [pltpu.VMEM((CH, D), F32) for _ in range(2)],
        pltpu.VMEM_SHARED((N, D), F32),
        [pltpu.SemaphoreType.DMA for _ in range(2)],
        [pltpu.SemaphoreType.DMA for _ in range(2)],
        [pltpu.SemaphoreType.DMA for _ in range(2)],
    ],
)
def _sc_scatter(dst, exwg, exwn, pg, pn, z128,
                wv_o, den_o,
                idx, buf, acc, sem_i, sem_r, sem_s):
    c = lax.axis_index("c")
    s = lax.axis_index("s")

    @pl.when(s < 15)
    def _():
        pltpu.sync_copy(z128.at[pl.ds(s * ROW_SPAN, ROW_SPAN)],
                        acc.at[pl.ds(s * ROW_SPAN, ROW_SPAN)])

    @pl.when(s == 15)
    def _():
        pltpu.sync_copy(z128.at[pl.ds(15 * ROW_SPAN, ROW_SPAN_LAST)],
                        acc.at[pl.ds(15 * ROW_SPAN, ROW_SPAN_LAST)])

    plsc.subcore_barrier()

    def run(rows_g, rows_n):
        def start_loads(k, b):
            t = s + 16 * k

            @pl.when(t < NCH_ALL)
            def _():
                pltpu.async_copy(dst.at[pl.ds(t * CH, CH)], idx[b], sem_i[b])

                @pl.when(t < NCH_HALF)
                def _():
                    pltpu.async_copy(rows_g.at[pl.ds(t * CH, CH)], buf[b], sem_r[b])

                @pl.when(t >= NCH_HALF)
                def _():
                    pltpu.async_copy(rows_n.at[pl.ds((t - NCH_HALF) * CH, CH)],
                                     buf[b], sem_r[b])

        start_loads(0, 0)

        def step(k, b):
            t = s + 16 * k

            @pl.when(t < NCH_ALL)
            def _():
                pltpu.make_async_copy(dst.at[pl.ds(0, CH)], idx[b], sem_i[b]).wait()
                pltpu.make_async_copy(rows_g.at[pl.ds(0, CH)], buf[b], sem_r[b]).wait()
                pltpu.async_copy(buf[b], acc.at[idx[b]], sem_s[b], add=True)

                @pl.when(k >= 1)
                def _():
                    pltpu.make_async_copy(buf[1 - b], acc.at[idx[1 - b]],
                                          sem_s[1 - b]).wait()

                start_loads(k + 1, 1 - b)

        def body(k2, z):
            step(2 * k2, 0)
            step(2 * k2 + 1, 1)
            return z

        lax.fori_loop(0, (NK_SCAT + 1) // 2, body, 0)

        t_last = s + 16 * (NK_SCAT - 1)
        b_last = (NK_SCAT - 1) % 2

        @pl.when(t_last < NCH_ALL)
        def _():
            pltpu.make_async_copy(buf[b_last], acc.at[idx[b_last]],
                                  sem_s[b_last]).wait()

        @pl.when(t_last >= NCH_ALL)
        def _():
            pltpu.make_async_copy(buf[1 - b_last], acc.at[idx[1 - b_last]],
                                  sem_s[1 - b_last]).wait()

    @pl.when(c == 0)
    def _():
        run(pg, pn)

    @pl.when(c == 1)
    def _():
        run(exwg, exwn)

    plsc.subcore_barrier()

    def flush(out):
        @pl.when(s < 15)
        def _():
            pltpu.sync_copy(acc.at[pl.ds(s * ROW_SPAN, ROW_SPAN)],
                            out.at[pl.ds(s * ROW_SPAN, ROW_SPAN)])

        @pl.when(s == 15)
        def _():
            pltpu.sync_copy(acc.at[pl.ds(15 * ROW_SPAN, ROW_SPAN_LAST)],
                            out.at[pl.ds(15 * ROW_SPAN, ROW_SPAN_LAST)])

    @pl.when(c == 0)
    def _():
        flush(wv_o)

    @pl.when(c == 1)
    def _():
        flush(den_o)


def _full(shape):
    nd = len(shape)
    return pl.BlockSpec(shape, lambda i: (0,) * nd)


def kernel(x, e, adj2, rel_pos_3d, edge_index_full, g_eids, params):
    p = params
    src = edge_index_full[0].astype(jnp.int32)
    dst = edge_index_full[1].astype(jnp.int32)

    r1 = lambda a: a.reshape(1, -1)

    qh, kh, vh = pl.pallas_call(
        _node_pro_body,
        out_shape=[jax.ShapeDtypeStruct((N, D), F32) for _ in range(3)],
    )(x, p['Wq'], r1(p['bq']), p['Wk'], r1(p['bk']), p['Wv'], r1(p['bv']),
      r1(p['gn1_w']), r1(p['gn1_b']), r1(p['gn1_a']))

    enorm, proj_e = pl.pallas_call(
        _edge_pro_body,
        grid=(NBLK_G,),
        in_specs=[pl.BlockSpec((BE, ED), lambda i: (i, 0)),
                  _full((1, ED)), _full((1, ED)),
                  _full((H, ED)), _full((1, H))],
        out_specs=[pl.BlockSpec((BE, ED), lambda i: (i, 0)),
                   pl.BlockSpec((BE, H), lambda i: (i, 0))],
        out_shape=[jax.ShapeDtypeStruct((EG, ED), F32),
                   jax.ShapeDtypeStruct((EG, H), F32)],
    )(e, r1(p['ln1e_w']), r1(p['ln1e_b']), p['Wpe'], r1(p['bpe']))

    ksg, qdg, vsg = _sc_gather_g(kh, qh, vh, src, dst)
    ksn, qdn, vsn = _sc_gather_n(kh, qh, vh, src, dst)

    eout, exwg, pg = pl.pallas_call(
        _score_g_body,
        grid=(NBLK_G,),
        in_specs=[pl.BlockSpec((BE, D), lambda i: (i, 0)),
                  pl.BlockSpec((BE, D), lambda i: (i, 0)),
                  pl.BlockSpec((BE, D), lambda i: (i, 0)),
                  pl.BlockSpec((BE, 1), lambda i: (i, 0)),
                  pl.BlockSpec((BE, H), lambda i: (i, 0)),
                  pl.BlockSpec((BE, H), lambda i: (i, 0)),
                  _full((ED, H)), _full((1, ED))],
        out_specs=[pl.BlockSpec((BE, ED), lambda i: (i, 0)),
                   pl.BlockSpec((BE, D), lambda i: (i, 0)),
                   pl.BlockSpec((BE, D), lambda i: (i, 0))],
        out_shape=[jax.ShapeDtypeStruct((EG, ED), F32),
                   jax.ShapeDtypeStruct((EG, D), F32),
                   jax.ShapeDtypeStruct((EG, D), F32)],
    )(ksg, qdg, vsg, adj2[:EG], rel_pos_3d[:EG], proj_e,
      p['Wap'], r1(p['bap']))

    exwn, pn = pl.pallas_call(
        _score_n_body,
        grid=(NBLK_G,),
        in_specs=[pl.BlockSpec((BE, D), lambda i: (i, 0)),
                  pl.BlockSpec((BE, D), lambda i: (i, 0)),
                  pl.BlockSpec((BE, D), lambda i: (i, 0)),
                  pl.BlockSpec((BE, 1), lambda i: (i, 0)),
                  pl.BlockSpec((BE, H), lambda i: (i, 0))],
        out_specs=[pl.BlockSpec((BE, D), lambda i: (i, 0)),
                   pl.BlockSpec((BE, D), lambda i: (i, 0))],
        out_shape=[jax.ShapeDtypeStruct((EG, D), F32),
                   jax.ShapeDtypeStruct((EG, D), F32)],
    )(ksn, qdn, vsn, adj2[EG:], rel_pos_3d[EG:])

    z128 = jnp.zeros((N, D), F32)
    wv, den = _sc_scatter(dst, exwg, exwn, pg, pn, z128)

    x2, mu, var = pl.pallas_call(
        _node_epi1_body,
        out_shape=[jax.ShapeDtypeStruct((N, D), F32),
                   jax.ShapeDtypeStruct((1, D), F32),
                   jax.ShapeDtypeStruct((1, D), F32)],
    )(x, wv, den, p['Wo'], r1(p['bo']), r1(p['gn2_a']))

    x3 = pl.pallas_call(
        _node_epi2_body,
        grid=(5,),
        in_specs=[pl.BlockSpec((BE, D), lambda i: (i, 0)),
                  _full((1, D)), _full((1, D)),
                  _full((1, D)), _full((1, D)), _full((1, D)),
                  _full((FFN, D)), _full((1, FFN)),
                  _full((D, FFN)), _full((1, D))],
        out_specs=pl.BlockSpec((BE, D), lambda i: (i, 0)),
        out_shape=jax.ShapeDtypeStruct((N, D), F32),
    )(x2, mu, var, r1(p['gn2_w']), r1(p['gn2_b']), r1(p['gn2_a']),
      p['Wh1'], r1(p['bh1']), p['Wh2'], r1(p['bh2']))

    e3 = pl.pallas_call(
        _edge_epi_body,
        grid=(NBLK_G,),
        in_specs=[pl.BlockSpec((BE, ED), lambda i: (i, 0)),
                  pl.BlockSpec((BE, ED), lambda i: (i, 0)),
                  pl.BlockSpec((BE, ED), lambda i: (i, 0)),
                  _full((ED, ED)), _full((1, ED)),
                  _full((1, ED)), _full((1, ED)),
                  _full((FFN, ED)), _full((1, FFN)),
                  _full((ED, FFN)), _full((1, ED))],
        out_specs=pl.BlockSpec((BE, ED), lambda i: (i, 0)),
        out_shape=jax.ShapeDtypeStruct((EG, ED), F32),
    )(e, enorm, eout, p['Woe'], r1(p['boe']),
      r1(p['ln2e_w']), r1(p['ln2e_b']),
      p['We1'], r1(p['be1']), p['We2'], r1(p['be2']))

    return (x3, e3)


# speedup vs baseline: 1.0982x; 1.0174x over previous
import functools

import jax
import jax.numpy as jnp
from jax import lax
from jax.experimental import pallas as pl
from jax.experimental.pallas import tpu as pltpu
from jax.experimental.pallas import tpu_sc as plsc

N = 10000
D = 128
H = 8
DH = 16
ED = 16
FFN = 512
E_FULL = 320000
EG = 160000

F32 = jnp.float32

CH = 128
NCH_HALF = EG // CH
NK_GATHER = (NCH_HALF + 15) // 16
NCH_ALL = E_FULL // CH
NK_SCAT = (NCH_ALL + 15) // 16
ROW_SPAN = 624
ROW_SPAN_LAST = N - 15 * ROW_SPAN

BE = 4000
NBLK_G = EG // BE
BN = 2000


def _dot(a, b, dims):
    return lax.dot_general(a, b, (dims, ((), ())),
                           preferred_element_type=F32)


def _sel_128x8():
    r = lax.broadcasted_iota(jnp.int32, (D, H), 0) // DH
    c = lax.broadcasted_iota(jnp.int32, (D, H), 1)
    return (r == c).astype(F32)


def _node_pro_body(x, wq, bq, wk, bk, wv, bv, gw, gb, ga, qh, kh, vh):
    xv = x[...]
    mean = jnp.mean(xv, axis=0, keepdims=True)
    sub = xv - ga[...] * mean
    var = jnp.mean(sub * sub, axis=0, keepdims=True)
    y = gw[...] * sub * lax.rsqrt(var + 1e-5) + gb[...]
    qh[...] = _dot(y, wq[...], ((1,), (1,))) + bq[...]
    kh[...] = _dot(y, wk[...], ((1,), (1,))) + bk[...]
    vh[...] = _dot(y, wv[...], ((1,), (1,))) + bv[...]


def _edge_pro_body(e, lw, lb, wpe, bpe, enorm, proj):
    ev = e[...]
    m = jnp.mean(ev, axis=1, keepdims=True)
    v = jnp.mean((ev - m) ** 2, axis=1, keepdims=True)
    en = lw[...] * (ev - m) * lax.rsqrt(v + 1e-5) + lb[...]
    enorm[...] = en
    proj[...] = _dot(en, wpe[...], ((1,), (1,))) + bpe[...]


def _score_g_body(ks, qd, vs, adj, rel, proj, wap, bap, eout, exw_o, p):
    sel = _sel_128x8()
    t = ks[...] * qd[...]
    dot = _dot(t, sel, ((1,), (0,)))
    score = dot * (adj[...] * 0.25) + rel[...]
    eout[...] = _dot(score, wap[...], ((1,), (1,))) + bap[...]
    score = score + proj[...]
    ex = jnp.exp(jnp.clip(score, -5.0, 5.0))
    exw = _dot(ex, sel, ((1,), (1,)))
    exw_o[...] = exw
    p[...] = vs[...] * exw


def _score_n_body(ks, qd, vs, adj, rel, exw_o, p):
    sel = _sel_128x8()
    t = ks[...] * qd[...]
    dot = _dot(t, sel, ((1,), (0,)))
    score = dot * (adj[...] * 0.25) + rel[...]
    ex = jnp.exp(jnp.clip(score, -5.0, 5.0))
    exw = _dot(ex, sel, ((1,), (1,)))
    exw_o[...] = exw
    p[...] = vs[...] * exw


def _node_epi1_body(x, wv, den, wo, bo, ga, x2, mu, var):
    denw = den[...] + 1e-16
    wvn_ = wv[...] / denw
    h_out = _dot(wvn_, wo[...], ((1,), (1,))) + bo[...]
    x2v = x[...] + h_out
    x2[...] = x2v
    m = jnp.mean(x2v, axis=0, keepdims=True)
    sub = x2v - ga[...] * m
    mu[...] = m
    var[...] = jnp.mean(sub * sub, axis=0, keepdims=True)


def _node_epi2_body(x2, mu, var, gw, gb, ga, w1, b1, w2, b2, x3):
    x2v = x2[...]
    sub = x2v - ga[...] * mu[...]
    y2 = gw[...] * sub * lax.rsqrt(var[...] + 1e-5) + gb[...]
    h = jax.nn.gelu(_dot(y2, w1[...], ((1,), (1,))) + b1[...])
    x3[...] = x2v + _dot(h, w2[...], ((1,), (1,))) + b2[...]


def _edge_epi_body(e, enorm, eout, woe, boe, lw, lb, w1, b1, w2, b2, e3):
    e_att = _dot(eout[...] + enorm[...], woe[...], ((1,), (1,))) + boe[...]
    e2 = e[...] + e_att
    m = jnp.mean(e2, axis=1, keepdims=True)
    v = jnp.mean((e2 - m) ** 2, axis=1, keepdims=True)
    en2 = lw[...] * (e2 - m) * lax.rsqrt(v + 1e-5) + lb[...]
    h = jax.nn.gelu(_dot(en2, w1[...], ((1,), (1,))) + b1[...])
    e3[...] = e2 + _dot(h, w2[...], ((1,), (1,))) + b2[...]


_MESH = plsc.VectorSubcoreMesh(core_axis_name="c", subcore_axis_name="s")


NCH_CORE = NCH_HALF // 2
NK_GATHER2 = (NCH_CORE + 15) // 16


def _make_gather(half_base):
    @functools.partial(
        pl.kernel,
        out_type=[jax.ShapeDtypeStruct((EG, D), F32) for _ in range(3)],
        mesh=_MESH,
        scratch_types=[
            [pltpu.VMEM((CH,), jnp.int32) for _ in range(2)],
            [pltpu.VMEM((CH,), jnp.int32) for _ in range(2)],
            [pltpu.VMEM((CH, D), F32) for _ in range(2)],
            [pltpu.VMEM((CH, D), F32) for _ in range(2)],
            [pltpu.VMEM((CH, D), F32) for _ in range(2)],
            [pltpu.SemaphoreType.DMA for _ in range(2)],
            [pltpu.SemaphoreType.DMA for _ in range(2)],
            [pltpu.SemaphoreType.DMA for _ in range(2)],
        ],
    )
    def gather(kh, qh, vh, src, dst, ks_o, qd_o, vs_o,
               idx_s, idx_d, rk, rq, rv, sem_i, sem_g, sem_w):
        c = lax.axis_index("c")
        s = lax.axis_index("s")

        def start_idx(k, b):
            tl = s + 16 * k

            @pl.when(tl < NCH_CORE)
            def _():
                base = half_base + (c * NCH_CORE + tl) * CH
                pltpu.async_copy(src.at[pl.ds(base, CH)], idx_s[b], sem_i[b])
                pltpu.async_copy(dst.at[pl.ds(base, CH)], idx_d[b], sem_i[b])

        def drain_w(b):
            pltpu.make_async_copy(rk[b], ks_o.at[pl.ds(0, CH)], sem_w[b]).wait()
            pltpu.make_async_copy(rq[b], qd_o.at[pl.ds(0, CH)], sem_w[b]).wait()
            pltpu.make_async_copy(rv[b], vs_o.at[pl.ds(0, CH)], sem_w[b]).wait()

        start_idx(0, 0)

        def step(k, b):
            tl = s + 16 * k

            @pl.when(tl < NCH_CORE)
            def _():
                loc = (c * NCH_CORE + tl) * CH
                pltpu.make_async_copy(src.at[pl.ds(0, CH)], idx_s[b], sem_i[b]).wait()
                pltpu.make_async_copy(dst.at[pl.ds(0, CH)], idx_d[b], sem_i[b]).wait()

                @pl.when(k >= 2)
                def _():
                    drain_w(b)

                pltpu.async_copy(kh.at[idx_s[b]], rk[b], sem_g[b])
                pltpu.async_copy(qh.at[idx_d[b]], rq[b], sem_g[b])
                pltpu.async_copy(vh.at[idx_s[b]], rv[b], sem_g[b])
                start_idx(k + 1, 1 - b)
                pltpu.make_async_copy(kh.at[idx_s[b]], rk[b], sem_g[b]).wait()
                pltpu.make_async_copy(qh.at[idx_d[b]], rq[b], sem_g[b]).wait()
                pltpu.make_async_copy(vh.at[idx_s[b]], rv[b], sem_g[b]).wait()
                pltpu.async_copy(rk[b], ks_o.at[pl.ds(loc, CH)], sem_w[b])
                pltpu.async_copy(rq[b], qd_o.at[pl.ds(loc, CH)], sem_w[b])
                pltpu.async_copy(rv[b], vs_o.at[pl.ds(loc, CH)], sem_w[b])

        def body(k2, z):
            step(2 * k2, 0)
            step(2 * k2 + 1, 1)
            return z

        lax.fori_loop(0, (NK_GATHER2 + 1) // 2, body, 0)

        def drain_tail(k):
            b = k % 2
            tl = s + 16 * k
            tl2 = s + 16 * (k + 2)

            @pl.when((tl < NCH_CORE) & (tl2 >= NCH_CORE))
            def _():
                drain_w(b)

        drain_tail(NK_GATHER2 - 3)
        drain_tail(NK_GATHER2 - 2)
        drain_tail(NK_GATHER2 - 1)

    return gather


_sc_gather_g = _make_gather(0)
_sc_gather_n = _make_gather(EG)


@functools.partial(
    pl.kernel,
    out_type=[
        jax.ShapeDtypeStruct((N, D), F32),
        jax.ShapeDtypeStruct((N, D), F32),
    ],
    mesh=_MESH,
    scratch_types=[
        [pltpu.VMEM((CH,), jnp.int32) for _ in range(2)],
        [pltpu.VMEM((CH, D), F32) for _ in range(2)],
        pltpu.VMEM_SHARED((N, D), F32),
        [pltpu.SemaphoreType.DMA for _ in range(2)],
        [pltpu.SemaphoreType.DMA for _ in range(2)],
        [pltpu.SemaphoreType.DMA for _ in range(2)],
    ],
)
def _sc_scatter(dst, exwg, exwn, pg, pn, z128,
                wv_o, den_o,
                idx, buf, acc, sem_i, sem_r, sem_s):
    c = lax.axis_index("c")
    s = lax.axis_index("s")

    @pl.when(s < 15)
    def _():
        pltpu.sync_copy(z128.at[pl.ds(s * ROW_SPAN, ROW_SPAN)],
                        acc.at[pl.ds(s * ROW_SPAN, ROW_SPAN)])

    @pl.when(s == 15)
    def _():
        pltpu.sync_copy(z128.at[pl.ds(15 * ROW_SPAN, ROW_SPAN_LAST)],
                        acc.at[pl.ds(15 * ROW_SPAN, ROW_SPAN_LAST)])

    plsc.subcore_barrier()

    def run(rows_g, rows_n):
        def start_loads(k, b):
            t = s + 16 * k

            @pl.when(t < NCH_ALL)
            def _():
                pltpu.async_copy(dst.at[pl.ds(t * CH, CH)], idx[b], sem_i[b])

                @pl.when(t < NCH_HALF)
                def _():
                    pltpu.async_copy(rows_g.at[pl.ds(t * CH, CH)], buf[b], sem_r[b])

                @pl.when(t >= NCH_HALF)
                def _():
                    pltpu.async_copy(rows_n.at[pl.ds((t - NCH_HALF) * CH, CH)],
                                     buf[b], sem_r[b])

        start_loads(0, 0)

        def step(k, b):
            t = s + 16 * k

            @pl.when(t < NCH_ALL)
            def _():
                pltpu.make_async_copy(dst.at[pl.ds(0, CH)], idx[b], sem_i[b]).wait()
                pltpu.make_async_copy(rows_g.at[pl.ds(0, CH)], buf[b], sem_r[b]).wait()
                pltpu.async_copy(buf[b], acc.at[idx[b]], sem_s[b], add=True)

                @pl.when(k >= 1)
                def _():
                    pltpu.make_async_copy(buf[1 - b], acc.at[idx[1 - b]],
                                          sem_s[1 - b]).wait()

                start_loads(k + 1, 1 - b)

        def body(k2, z):
            step(2 * k2, 0)
            step(2 * k2 + 1, 1)
            return z

        lax.fori_loop(0, (NK_SCAT + 1) // 2, body, 0)

        t_last = s + 16 * (NK_SCAT - 1)
        b_last = (NK_SCAT - 1) % 2

        @pl.when(t_last < NCH_ALL)
        def _():
            pltpu.make_async_copy(buf[b_last], acc.at[idx[b_last]],
                                  sem_s[b_last]).wait()

        @pl.when(t_last >= NCH_ALL)
        def _():
            pltpu.make_async_copy(buf[1 - b_last], acc.at[idx[1 - b_last]],
                                  sem_s[1 - b_last]).wait()

    @pl.when(c == 0)
    def _():
        run(pg, pn)

    @pl.when(c == 1)
    def _():
        run(exwg, exwn)

    plsc.subcore_barrier()

    def flush(out):
        @pl.when(s < 15)
        def _():
            pltpu.sync_copy(acc.at[pl.ds(s * ROW_SPAN, ROW_SPAN)],
                            out.at[pl.ds(s * ROW_SPAN, ROW_SPAN)])

        @pl.when(s == 15)
        def _():
            pltpu.sync_copy(acc.at[pl.ds(15 * ROW_SPAN, ROW_SPAN_LAST)],
                            out.at[pl.ds(15 * ROW_SPAN, ROW_SPAN_LAST)])

    @pl.when(c == 0)
    def _():
        flush(wv_o)

    @pl.when(c == 1)
    def _():
        flush(den_o)


def _full(shape):
    nd = len(shape)
    return pl.BlockSpec(shape, lambda i: (0,) * nd)


def kernel(x, e, adj2, rel_pos_3d, edge_index_full, g_eids, params):
    p = params
    src = edge_index_full[0].astype(jnp.int32)
    dst = edge_index_full[1].astype(jnp.int32)

    r1 = lambda a: a.reshape(1, -1)

    qh, kh, vh = pl.pallas_call(
        _node_pro_body,
        out_shape=[jax.ShapeDtypeStruct((N, D), F32) for _ in range(3)],
    )(x, p['Wq'], r1(p['bq']), p['Wk'], r1(p['bk']), p['Wv'], r1(p['bv']),
      r1(p['gn1_w']), r1(p['gn1_b']), r1(p['gn1_a']))

    enorm, proj_e = pl.pallas_call(
        _edge_pro_body,
        grid=(NBLK_G,),
        in_specs=[pl.BlockSpec((BE, ED), lambda i: (i, 0)),
                  _full((1, ED)), _full((1, ED)),
                  _full((H, ED)), _full((1, H))],
        out_specs=[pl.BlockSpec((BE, ED), lambda i: (i, 0)),
                   pl.BlockSpec((BE, H), lambda i: (i, 0))],
        out_shape=[jax.ShapeDtypeStruct((EG, ED), F32),
                   jax.ShapeDtypeStruct((EG, H), F32)],
    )(e, r1(p['ln1e_w']), r1(p['ln1e_b']), p['Wpe'], r1(p['bpe']))

    ksg, qdg, vsg = _sc_gather_g(kh, qh, vh, src, dst)
    ksn, qdn, vsn = _sc_gather_n(kh, qh, vh, src, dst)

    eout, exwg, pg = pl.pallas_call(
        _score_g_body,
        grid=(NBLK_G,),
        in_specs=[pl.BlockSpec((BE, D), lambda i: (i, 0)),
                  pl.BlockSpec((BE, D), lambda i: (i, 0)),
                  pl.BlockSpec((BE, D), lambda i: (i, 0)),
                  pl.BlockSpec((BE, 1), lambda i: (i, 0)),
                  pl.BlockSpec((BE, H), lambda i: (i, 0)),
                  pl.BlockSpec((BE, H), lambda i: (i, 0)),
                  _full((ED, H)), _full((1, ED))],
        out_specs=[pl.BlockSpec((BE, ED), lambda i: (i, 0)),
                   pl.BlockSpec((BE, D), lambda i: (i, 0)),
                   pl.BlockSpec((BE, D), lambda i: (i, 0))],
        out_shape=[jax.ShapeDtypeStruct((EG, ED), F32),
                   jax.ShapeDtypeStruct((EG, D), F32),
                   jax.ShapeDtypeStruct((EG, D), F32)],
    )(ksg, qdg, vsg, adj2[:EG], rel_pos_3d[:EG], proj_e,
      p['Wap'], r1(p['bap']))

    exwn, pn = pl.pallas_call(
        _score_n_body,
        grid=(NBLK_G,),
        in_specs=[pl.BlockSpec((BE, D), lambda i: (i, 0)),
                  pl.BlockSpec((BE, D), lambda i: (i, 0)),
                  pl.BlockSpec((BE, D), lambda i: (i, 0)),
                  pl.BlockSpec((BE, 1), lambda i: (i, 0)),
                  pl.BlockSpec((BE, H), lambda i: (i, 0))],
        out_specs=[pl.BlockSpec((BE, D), lambda i: (i, 0)),
                   pl.BlockSpec((BE, D), lambda i: (i, 0))],
        out_shape=[jax.ShapeDtypeStruct((EG, D), F32),
                   jax.ShapeDtypeStruct((EG, D), F32)],
    )(ksn, qdn, vsn, adj2[EG:], rel_pos_3d[EG:])

    z128 = jnp.zeros((N, D), F32)
    wv, den = _sc_scatter(dst, exwg, exwn, pg, pn, z128)

    x2, mu, var = pl.pallas_call(
        _node_epi1_body,
        out_shape=[jax.ShapeDtypeStruct((N, D), F32),
                   jax.ShapeDtypeStruct((1, D), F32),
                   jax.ShapeDtypeStruct((1, D), F32)],
    )(x, wv, den, p['Wo'], r1(p['bo']), r1(p['gn2_a']))

    x3 = pl.pallas_call(
        _node_epi2_body,
        grid=(5,),
        in_specs=[pl.BlockSpec((BN, D), lambda i: (i, 0)),
                  _full((1, D)), _full((1, D)),
                  _full((1, D)), _full((1, D)), _full((1, D)),
                  _full((FFN, D)), _full((1, FFN)),
                  _full((D, FFN)), _full((1, D))],
        out_specs=pl.BlockSpec((BN, D), lambda i: (i, 0)),
        out_shape=jax.ShapeDtypeStruct((N, D), F32),
    )(x2, mu, var, r1(p['gn2_w']), r1(p['gn2_b']), r1(p['gn2_a']),
      p['Wh1'], r1(p['bh1']), p['Wh2'], r1(p['bh2']))

    e3 = pl.pallas_call(
        _edge_epi_body,
        grid=(NBLK_G,),
        in_specs=[pl.BlockSpec((BE, ED), lambda i: (i, 0)),
                  pl.BlockSpec((BE, ED), lambda i: (i, 0)),
                  pl.BlockSpec((BE, ED), lambda i: (i, 0)),
                  _full((ED, ED)), _full((1, ED)),
                  _full((1, ED)), _full((1, ED)),
                  _full((FFN, ED)), _full((1, FFN)),
                  _full((ED, FFN)), _full((1, ED))],
        out_specs=pl.BlockSpec((BE, ED), lambda i: (i, 0)),
        out_shape=jax.ShapeDtypeStruct((EG, ED), F32),
    )(e, enorm, eout, p['Woe'], r1(p['boe']),
      r1(p['ln2e_w']), r1(p['ln2e_b']),
      p['We1'], r1(p['be1']), p['We2'], r1(p['be2']))

    return (x3, e3)
